# Initial kernel scaffold; baseline (speedup 1.0000x reference)
import jax
import jax.numpy as jnp
from jax.experimental import pallas as pl


def kernel(x, edge_index, theta_direct, theta_heat1, theta_hidden, theta_heat2, t):
    raise NotImplementedError("write your pallas kernel here")


# speedup vs baseline: 4.7147x; 4.7147x over previous
import functools

import jax
import jax.numpy as jnp
from jax import lax
from jax.scipy.special import gammaln
from jax.experimental import pallas as pl
from jax.experimental.pallas import tpu as pltpu
from jax.experimental.pallas import tpu_sc as plsc

N = 10000
E = 320000
D = 128
K = 10

NC = 2
NS = 16
NW = NC * NS
CH = 128
EPW = E // NW
NCH = -(-EPW // CH)
EPW_PAD = NCH * CH
ROWS_PER_TILE = 632
NACC = ROWS_PER_TILE * NS


def _sc_lap_body(x_hbm, col_hbm, row_hbm, out_hbm, colv, rowv, gbuf, acc):
    c = lax.axis_index("c")
    s = lax.axis_index("s")
    w = c * NS + s
    pltpu.sync_copy(col_hbm.at[w], colv)
    pltpu.sync_copy(row_hbm.at[w], rowv)

    def _z(i, carry):
        gbuf[i // 8, pl.ds((i % 8) * 16, 16)] = jnp.zeros((16,), jnp.float32)
        return carry

    lax.fori_loop(0, CH * 8, _z, 0)
    base = s * ROWS_PER_TILE
    for r in range(ROWS_PER_TILE // CH):
        pltpu.sync_copy(gbuf, acc.at[pl.ds(base + r * CH, CH)])
    rem = ROWS_PER_TILE % CH
    if rem:
        pltpu.sync_copy(
            gbuf.at[pl.ds(0, rem)],
            acc.at[pl.ds(base + (ROWS_PER_TILE // CH) * CH, rem)],
        )
    plsc.subcore_barrier()

    def _step(j, carry):
        pltpu.sync_copy(x_hbm.at[colv.at[j]], gbuf)
        pltpu.sync_copy(gbuf, acc.at[rowv.at[j]], add=True)
        return carry

    lax.fori_loop(0, NCH, _step, 0)
    plsc.subcore_barrier()

    pltpu.sync_copy(
        acc.at[pl.ds(base, ROWS_PER_TILE)],
        out_hbm.at[c, pl.ds(base, ROWS_PER_TILE)],
    )


_sc_lap = functools.partial(
    pl.kernel,
    mesh=plsc.VectorSubcoreMesh(core_axis_name="c", subcore_axis_name="s"),
    out_type=jax.ShapeDtypeStruct((NC, NACC, D), jnp.float32),
    scratch_types=[
        pltpu.VMEM((NCH, CH), jnp.int32),
        pltpu.VMEM((NCH, CH), jnp.int32),
        pltpu.VMEM((CH, D), jnp.float32),
        pltpu.VMEM_SHARED((NACC, D), jnp.float32),
    ],
)(_sc_lap_body)


def _dense_relu_body(x_ref, xh_ref, w1_ref, w2_ref, o_ref):
    acc = jnp.dot(x_ref[...], w1_ref[...], preferred_element_type=jnp.float32)
    acc = acc + jnp.dot(xh_ref[...], w2_ref[...], preferred_element_type=jnp.float32)
    o_ref[...] = jnp.maximum(acc, 0.0)


def _dense_lsm_body(x_ref, xh_ref, w1_ref, w2_ref, o_ref):
    acc = jnp.dot(x_ref[...], w1_ref[...], preferred_element_type=jnp.float32)
    acc = acc + jnp.dot(xh_ref[...], w2_ref[...], preferred_element_type=jnp.float32)
    m = jnp.max(acc, axis=1, keepdims=True)
    ex = jnp.exp(acc - m)
    lse = jnp.log(jnp.sum(ex, axis=1, keepdims=True)) + m
    o_ref[...] = acc - lse


_BLK = 1000


def _dense_call(body, x, xh, w1, w2):
    return pl.pallas_call(
        body,
        grid=(N // _BLK,),
        in_specs=[
            pl.BlockSpec((_BLK, D), lambda i: (i, 0)),
            pl.BlockSpec((_BLK, D), lambda i: (i, 0)),
            pl.BlockSpec((D, D), lambda i: (0, 0)),
            pl.BlockSpec((D, D), lambda i: (0, 0)),
        ],
        out_specs=pl.BlockSpec((_BLK, D), lambda i: (i, 0)),
        out_shape=jax.ShapeDtypeStruct((N, D), jnp.float32),
    )(x, xh, w1, w2)


def _iv(v, x):
    m = jnp.arange(30.0, dtype=jnp.float32)
    log_terms = (2.0 * m + v) * jnp.log(x / 2.0) - gammaln(m + 1.0) - gammaln(m + v + 1.0)
    return jnp.sum(jnp.exp(log_terms))


def kernel(x, edge_index, theta_direct, theta_heat1, theta_hidden, theta_heat2, t):
    row = edge_index[0]
    col = edge_index[1]
    pad = NW * EPW_PAD - E
    colp = jnp.concatenate([col, jnp.zeros((pad,), jnp.int32)]).reshape(NW, NCH, CH)
    rowp = jnp.concatenate([row, jnp.full((pad,), N, jnp.int32)]).reshape(NW, NCH, CH)

    ones = jnp.ones((N, D), jnp.float32)
    degp = _sc_lap(ones, colp, rowp)
    deg = degp[0, :N, 0] + degp[1, :N, 0]
    dis = jnp.where(deg > 0, lax.rsqrt(jnp.maximum(deg, 1e-12)), 0.0)[:, None]

    def lap(v):
        p = _sc_lap(dis * v, colp, rowp)
        return -dis * (p[0, :N] + p[1, :N])

    def heat(v):
        out = _iv(0.0, t) * v
        t1 = lap(v)
        out = out - 2.0 * _iv(1.0, t) * t1
        tkm2, tkm1 = v, t1
        for k in range(2, K):
            tk = 2.0 * lap(tkm1) - tkm2
            out = out + 2.0 * ((-1.0) ** k) * _iv(float(k), t) * tk
            tkm2, tkm1 = tkm1, tk
        return out

    xh = heat(x)
    hidden = _dense_call(_dense_relu_body, x, xh, theta_direct, theta_heat1)
    hh = heat(hidden)
    return _dense_call(_dense_lsm_body, hidden, hh, theta_hidden, theta_heat2)
